# Initial kernel scaffold; baseline (speedup 1.0000x reference)
import jax
import jax.numpy as jnp
from jax.experimental import pallas as pl


def kernel(node_features, edge_features, edge_index, Wn, bn, We, be, W1s, W1n, b1, W2s, W2n, b2, Wp, bp):
    raise NotImplementedError("write your pallas kernel here")


# speedup vs baseline: 5.8804x; 5.8804x over previous
import functools

import jax
import jax.numpy as jnp
from jax import lax
from jax.experimental import pallas as pl
from jax.experimental.pallas import tpu as pltpu
from jax.experimental.pallas import tpu_sc as plsc

N = 10000
E = 320000
D = 128

CH = 80
NCHUNK = E // CH
CPT = NCHUNK // 16
CPT2 = NCHUNK // 32
RPT = N // 16
ZR = 125
EPT = E // 32


def _sc_mesh():
    return plsc.VectorSubcoreMesh(core_axis_name="c", subcore_axis_name="s")


def _edge_encode(xe, we, be, wp3, bp):
    BE = 8000

    def body(xe_ref, we_ref, be_ref, wp3_ref, bp_ref, e_ref, pe_ref):
        e = jnp.maximum(
            jnp.dot(xe_ref[...], we_ref[...],
                    preferred_element_type=jnp.float32) + be_ref[...], 0.0)
        e_ref[...] = e
        pe_ref[...] = jnp.dot(e, wp3_ref[...],
                              preferred_element_type=jnp.float32) + bp_ref[...]

    return pl.pallas_call(
        body,
        grid=(E // BE,),
        in_specs=[
            pl.BlockSpec((BE, D), lambda i: (i, 0)),
            pl.BlockSpec((D, D), lambda i: (0, 0)),
            pl.BlockSpec((1, D), lambda i: (0, 0)),
            pl.BlockSpec((D, 1), lambda i: (0, 0)),
            pl.BlockSpec((1, 1), lambda i: (0, 0)),
        ],
        out_specs=[
            pl.BlockSpec((BE, D), lambda i: (i, 0)),
            pl.BlockSpec((BE, 1), lambda i: (i, 0)),
        ],
        out_shape=[
            jax.ShapeDtypeStruct((E, D), jnp.float32),
            jax.ShapeDtypeStruct((E, 1), jnp.float32),
        ],
    )(xe, we, be.reshape(1, D), wp3, bp.reshape(1, 1))


def _node_encode(xn, wn, bn):
    def body(x_ref, w_ref, b_ref, o_ref):
        o_ref[...] = jnp.maximum(
            jnp.dot(x_ref[...], w_ref[...],
                    preferred_element_type=jnp.float32) + b_ref[...], 0.0)

    return pl.pallas_call(
        body,
        out_shape=jax.ShapeDtypeStruct((N, D), jnp.float32),
    )(xn, wn, bn.reshape(1, D))


def _layer1(n, sn, se, deg, w1s, w1n, b1):
    def body(n_ref, sn_ref, se_ref, deg_ref, ws_ref, wn_ref, b_ref, h_ref):
        agg = (sn_ref[...] + se_ref[...]) / jnp.maximum(deg_ref[...], 1.0)
        h_ref[...] = jnp.maximum(
            jnp.dot(n_ref[...], ws_ref[...], preferred_element_type=jnp.float32)
            + jnp.dot(agg, wn_ref[...], preferred_element_type=jnp.float32)
            + b_ref[...], 0.0)

    return pl.pallas_call(
        body,
        out_shape=jax.ShapeDtypeStruct((N, D), jnp.float32),
    )(n, sn, se, deg.reshape(N, 1), w1s, w1n, b1.reshape(1, D))


def _layer2_predict(h, sha, shb, se, deg, w2s, w2n, b2, wp12):
    def body(h_ref, sa_ref, sb_ref, se_ref, deg_ref, ws_ref, wn_ref, b_ref,
             wp_ref, p_ref):
        agg = (sa_ref[...] + sb_ref[...] + se_ref[...]) / jnp.maximum(
            deg_ref[...], 1.0)
        h2 = (jnp.dot(h_ref[...], ws_ref[...],
                      preferred_element_type=jnp.float32)
              + jnp.dot(agg, wn_ref[...], preferred_element_type=jnp.float32)
              + b_ref[...])
        p_ref[...] = jnp.dot(h2, wp_ref[...],
                             preferred_element_type=jnp.float32)

    return pl.pallas_call(
        body,
        out_shape=jax.ShapeDtypeStruct((N, 2), jnp.float32),
    )(h, sha, shb, se, deg.reshape(N, 1), w2s, w2n, b2.reshape(1, D), wp12)


def _seg1(e, nfeat, src2d, dst2d, zrows, zdeg):
    @functools.partial(
        pl.kernel,
        out_type=[
            jax.ShapeDtypeStruct((N, D), jnp.float32),
            jax.ShapeDtypeStruct((N, D), jnp.float32),
            jax.ShapeDtypeStruct((N,), jnp.float32),
        ],
        mesh=_sc_mesh(),
        compiler_params=pltpu.CompilerParams(use_tc_tiling_on_sc=False),
        scratch_types=[
            pltpu.VMEM((CPT, CH), jnp.int32),
            pltpu.VMEM((CPT, CH), jnp.int32),
            pltpu.VMEM((CH, D), jnp.float32),
            pltpu.VMEM((CH,), jnp.float32),
            pltpu.VMEM_SHARED((N, D), jnp.float32),
            pltpu.VMEM_SHARED((N,), jnp.float32),
            pltpu.SemaphoreType.DMA,
        ],
    )
    def k(e_hbm, n_hbm, src_hbm, dst_hbm, zr_hbm, zd_hbm,
          se_hbm, sn_hbm, deg_hbm,
          sbuf, dbuf, rowbuf, ones, acc, degacc, sem):
        c = lax.axis_index("c")
        t = lax.axis_index("s")

        for r in range(RPT // ZR):
            pltpu.sync_copy(zr_hbm, acc.at[pl.ds(t * RPT + r * ZR, ZR)])

        @pl.when(jnp.logical_and(c == 0, t == 0))
        def _():
            pltpu.sync_copy(zd_hbm, degacc)

        pltpu.sync_copy(dst_hbm.at[pl.ds(t * CPT, CPT)], dbuf)

        @pl.when(c == 1)
        def _():
            pltpu.sync_copy(src_hbm.at[pl.ds(t * CPT, CPT)], sbuf)

        for j in range(CH // 16):
            ones[pl.ds(j * 16, 16)] = jnp.full((16,), 1.0, jnp.float32)

        plsc.subcore_barrier()

        base = t * CPT

        @pl.when(c == 0)
        def _():
            def body(j, carry):
                pltpu.sync_copy(e_hbm.at[pl.ds((base + j) * CH, CH)], rowbuf)
                pltpu.sync_copy(rowbuf, acc.at[dbuf.at[j]], add=True)
                pltpu.sync_copy(ones, degacc.at[dbuf.at[j]], add=True)
                return carry
            lax.fori_loop(0, CPT, body, 0)

        @pl.when(c == 1)
        def _():
            def body(j, carry):
                pltpu.async_copy(n_hbm.at[sbuf.at[j]], rowbuf, sem).wait()
                pltpu.sync_copy(rowbuf, acc.at[dbuf.at[j]], add=True)
                return carry
            lax.fori_loop(0, CPT, body, 0)

        plsc.subcore_barrier()

        @pl.when(c == 0)
        def _():
            pltpu.sync_copy(acc.at[pl.ds(t * RPT, RPT)],
                            se_hbm.at[pl.ds(t * RPT, RPT)])

            @pl.when(t == 0)
            def _():
                pltpu.sync_copy(degacc, deg_hbm)

        @pl.when(c == 1)
        def _():
            pltpu.sync_copy(acc.at[pl.ds(t * RPT, RPT)],
                            sn_hbm.at[pl.ds(t * RPT, RPT)])

    return k(e, nfeat, src2d, dst2d, zrows, zdeg)


def _seg2(h, src2d, dst2d, zrows):
    @functools.partial(
        pl.kernel,
        out_type=[
            jax.ShapeDtypeStruct((N, D), jnp.float32),
            jax.ShapeDtypeStruct((N, D), jnp.float32),
        ],
        mesh=_sc_mesh(),
        compiler_params=pltpu.CompilerParams(use_tc_tiling_on_sc=False),
        scratch_types=[
            pltpu.VMEM((CPT2, CH), jnp.int32),
            pltpu.VMEM((CPT2, CH), jnp.int32),
            pltpu.VMEM((CH, D), jnp.float32),
            pltpu.VMEM_SHARED((N, D), jnp.float32),
            pltpu.SemaphoreType.DMA,
        ],
    )
    def k(h_hbm, src_hbm, dst_hbm, zr_hbm, sha_hbm, shb_hbm,
          sbuf, dbuf, rowbuf, acc, sem):
        c = lax.axis_index("c")
        t = lax.axis_index("s")

        for r in range(RPT // ZR):
            pltpu.sync_copy(zr_hbm, acc.at[pl.ds(t * RPT + r * ZR, ZR)])

        cb = c * (NCHUNK // 2) + t * CPT2
        pltpu.sync_copy(src_hbm.at[pl.ds(cb, CPT2)], sbuf)
        pltpu.sync_copy(dst_hbm.at[pl.ds(cb, CPT2)], dbuf)

        plsc.subcore_barrier()

        base = cb

        def body(j, carry):
            pltpu.async_copy(h_hbm.at[sbuf.at[j]], rowbuf, sem).wait()
            pltpu.sync_copy(rowbuf, acc.at[dbuf.at[j]], add=True)
            return carry
        lax.fori_loop(0, CPT2, body, 0)

        plsc.subcore_barrier()

        @pl.when(c == 0)
        def _():
            pltpu.sync_copy(acc.at[pl.ds(t * RPT, RPT)],
                            sha_hbm.at[pl.ds(t * RPT, RPT)])

        @pl.when(c == 1)
        def _():
            pltpu.sync_copy(acc.at[pl.ds(t * RPT, RPT)],
                            shb_hbm.at[pl.ds(t * RPT, RPT)])

    return k(h, src2d, dst2d, zrows)


def _predict_edges(peb, p1, p2, srcf, dstf):
    @functools.partial(
        pl.kernel,
        out_type=jax.ShapeDtypeStruct((E,), jnp.float32),
        mesh=_sc_mesh(),
        compiler_params=pltpu.CompilerParams(use_tc_tiling_on_sc=False,
                                             needs_layout_passes=False),
        scratch_types=[
            pltpu.VMEM((N,), jnp.float32),
            pltpu.VMEM((N,), jnp.float32),
            pltpu.VMEM((EPT,), jnp.int32),
            pltpu.VMEM((EPT,), jnp.int32),
            pltpu.VMEM((EPT,), jnp.float32),
            pltpu.VMEM((EPT,), jnp.float32),
        ],
    )
    def k(peb_hbm, p1_hbm, p2_hbm, src_hbm, dst_hbm, out_hbm,
          p1b, p2b, sb, db, pb, ob):
        c = lax.axis_index("c")
        t = lax.axis_index("s")
        w = t * 2 + c
        base = w * EPT

        pltpu.sync_copy(p1_hbm, p1b)
        pltpu.sync_copy(p2_hbm, p2b)
        pltpu.sync_copy(src_hbm.at[pl.ds(base, EPT)], sb)
        pltpu.sync_copy(dst_hbm.at[pl.ds(base, EPT)], db)
        pltpu.sync_copy(peb_hbm.at[pl.ds(base, EPT)], pb)

        def body(i, carry):
            off = pl.multiple_of(i * 16, 16)
            si = sb[pl.ds(off, 16)]
            di = db[pl.ds(off, 16)]
            v = (plsc.load_gather(p1b, [si]) + plsc.load_gather(p2b, [di])
                 + pb[pl.ds(off, 16)])
            ob[pl.ds(off, 16)] = v
            return carry
        lax.fori_loop(0, EPT // 16, body, 0)

        pltpu.sync_copy(ob, out_hbm.at[pl.ds(base, EPT)])

    return k(peb, p1, p2, srcf, dstf)


def kernel(node_features, edge_features, edge_index, Wn, bn, We, be,
           W1s, W1n, b1, W2s, W2n, b2, Wp, bp):
    srcf = edge_index[0]
    dstf = edge_index[1]
    src2d = srcf.reshape(NCHUNK, CH)
    dst2d = dstf.reshape(NCHUNK, CH)

    wp1 = Wp[0:D]
    wp2 = Wp[D:2 * D]
    wp3 = Wp[2 * D:3 * D]
    wp12 = jnp.concatenate([wp1, wp2], axis=1)

    zrows = jnp.zeros((ZR, D), jnp.float32)
    zdeg = jnp.zeros((N,), jnp.float32)

    e, peb = _edge_encode(edge_features, We, be, wp3, bp)
    n = _node_encode(node_features, Wn, bn)

    se, sn, deg = _seg1(e, n, src2d, dst2d, zrows, zdeg)
    h = _layer1(n, sn, se, deg, W1s, W1n, b1)
    sha, shb = _seg2(h, src2d, dst2d, zrows)
    p = _layer2_predict(h, sha, shb, se, deg, W2s, W2n, b2, wp12)

    out = _predict_edges(peb.reshape(E), p[:, 0], p[:, 1], srcf, dstf)
    return out.reshape(E, 1)
